# Initial kernel scaffold; baseline (speedup 1.0000x reference)
import jax
import jax.numpy as jnp
from jax.experimental import pallas as pl


def kernel(x, edge_index, eps, W1, b1, g1, be1, W2, b2, g2, be2):
    raise NotImplementedError("write your pallas kernel here")


# speedup vs baseline: 3.0709x; 3.0709x over previous
import functools

import jax
import jax.numpy as jnp
from jax import lax
from jax.experimental import pallas as pl
from jax.experimental.pallas import tpu as pltpu
from jax.experimental.pallas import tpu_sc as plsc

N = 10000
E = 320000
D = 128

NT = 16
NW = 32
C = 128
NPAIR = 40
EPW = 2 * NPAIR * C
EPAD = NW * EPW
NP = 10240
RPT = NP // NT

_mesh = plsc.VectorSubcoreMesh(core_axis_name="c", subcore_axis_name="s")


@functools.partial(
    pl.kernel,
    mesh=_mesh,
    out_type=jax.ShapeDtypeStruct((2 * NP, D), jnp.float32),
    scratch_types=[
        pltpu.VMEM((4, C), jnp.int32),
        pltpu.VMEM((4, C), jnp.int32),
        pltpu.VMEM((C, D), jnp.float32),
        pltpu.VMEM((C, D), jnp.float32),
        pltpu.VMEM_SHARED((NP, D), jnp.float32),
        pltpu.SemaphoreType.DMA,
        pltpu.SemaphoreType.DMA,
        pltpu.SemaphoreType.DMA,
        pltpu.SemaphoreType.DMA,
    ],
)
def _sc_segment_sum(x_hbm, src_hbm, dst_hbm, zero_hbm, out_hbm,
                    sidx, didx, rows_a, rows_b, agg,
                    sem_a, sem_b, sem_si, sem_di):
    c = lax.axis_index("c")
    s = lax.axis_index("s")
    w = c * NT + s

    pltpu.sync_copy(zero_hbm.at[pl.ds(s * RPT, RPT)],
                    agg.at[pl.ds(s * RPT, RPT)])

    pltpu.sync_copy(src_hbm.at[w, 0], sidx.at[pl.ds(0, 2)])
    pltpu.sync_copy(dst_hbm.at[w, 0], didx.at[pl.ds(0, 2)])

    plsc.subcore_barrier()

    def body(i, carry):
        base = (i % 2) * 2
        nbase = 2 - base
        nxt = lax.min(i + 1, NPAIR - 1)
        cp_si = pltpu.async_copy(src_hbm.at[w, nxt],
                                 sidx.at[pl.ds(nbase, 2)], sem_si)
        cp_di = pltpu.async_copy(dst_hbm.at[w, nxt],
                                 didx.at[pl.ds(nbase, 2)], sem_di)
        cp_a = pltpu.async_copy(x_hbm.at[sidx.at[base]], rows_a, sem_a)
        cp_b = pltpu.async_copy(x_hbm.at[sidx.at[base + 1]], rows_b, sem_b)
        cp_a.wait()
        pltpu.sync_copy(rows_a, agg.at[didx.at[base]], add=True)
        cp_b.wait()
        pltpu.sync_copy(rows_b, agg.at[didx.at[base + 1]], add=True)
        cp_si.wait()
        cp_di.wait()
        return carry

    lax.fori_loop(0, NPAIR, body, 0)

    plsc.subcore_barrier()

    pltpu.sync_copy(agg.at[pl.ds(s * RPT, RPT)],
                    out_hbm.at[pl.ds(c * NP + s * RPT, RPT)])


def _mlp_block(h, W, b, g, be):
    h = lax.dot_general(h, W, (((1,), (1,)), ((), ())),
                        preferred_element_type=jnp.float32) + b
    m = jnp.mean(h, axis=0, keepdims=True)
    v = jnp.mean((h - m) ** 2, axis=0, keepdims=True)
    h = (h - m) / jnp.sqrt(v + 1e-5) * g + be
    return h * jax.nn.sigmoid(h)


def _tc_mlp_body(p0, p1, x, eps, W1, b1, g1, be1, W2, b2, g2, be2, o):
    h = p0[...] + p1[...] + (1.0 + eps[0, 0]) * x[...]
    h = _mlp_block(h, W1[...], b1[...], g1[...], be1[...])
    h = _mlp_block(h, W2[...], b2[...], g2[...], be2[...])
    o[...] = h


def kernel(x, edge_index, eps, W1, b1, g1, be1, W2, b2, g2, be2):
    src = edge_index[0].astype(jnp.int32)
    dst = edge_index[1].astype(jnp.int32)
    pad = EPAD - E
    src = jnp.concatenate([src, jnp.zeros((pad,), jnp.int32)])
    dst = jnp.concatenate([dst, jnp.full((pad,), N, jnp.int32)])
    src = src.reshape(NW, NPAIR, 2, C)
    dst = dst.reshape(NW, NPAIR, 2, C)
    zeros = jnp.zeros((NP, D), jnp.float32)

    partials = _sc_segment_sum(x, src, dst, zeros)
    p0 = partials[:N]
    p1 = partials[NP:NP + N]

    out = pl.pallas_call(
        _tc_mlp_body,
        out_shape=jax.ShapeDtypeStruct((N, D), jnp.float32),
    )(p0, p1, x, eps.reshape(1, 1),
      W1, b1.reshape(1, D), g1.reshape(1, D), be1.reshape(1, D),
      W2, b2.reshape(1, D), g2.reshape(1, D), be2.reshape(1, D))
    return out


# speedup vs baseline: 3.0812x; 1.0034x over previous
import functools

import jax
import jax.numpy as jnp
from jax import lax
from jax.experimental import pallas as pl
from jax.experimental.pallas import tpu as pltpu
from jax.experimental.pallas import tpu_sc as plsc

N = 10000
E = 320000
D = 128

NT = 16
NW = 32
C = 128
NPAIR = 40
EPW = 2 * NPAIR * C
EPAD = NW * EPW
NP = 10240
RPT = NP // NT

_mesh = plsc.VectorSubcoreMesh(core_axis_name="c", subcore_axis_name="s")


@functools.partial(
    pl.kernel,
    mesh=_mesh,
    out_type=jax.ShapeDtypeStruct((2 * NP, D), jnp.float32),
    scratch_types=[
        pltpu.VMEM((4, C), jnp.int32),
        pltpu.VMEM((4, C), jnp.int32),
        pltpu.VMEM((C, D), jnp.float32),
        pltpu.VMEM((C, D), jnp.float32),
        pltpu.VMEM_SHARED((NP, D), jnp.float32),
        pltpu.SemaphoreType.DMA,
        pltpu.SemaphoreType.DMA,
        pltpu.SemaphoreType.DMA,
        pltpu.SemaphoreType.DMA,
        pltpu.SemaphoreType.DMA,
        pltpu.SemaphoreType.DMA,
    ],
)
def _sc_segment_sum(x_hbm, src_hbm, dst_hbm, zero_hbm, out_hbm,
                    sidx, didx, rows_a, rows_b, agg,
                    sem_a, sem_b, sem_si, sem_di, sem_sa, sem_sb):
    c = lax.axis_index("c")
    s = lax.axis_index("s")
    w = c * NT + s

    pltpu.sync_copy(zero_hbm.at[pl.ds(s * RPT, RPT)],
                    agg.at[pl.ds(s * RPT, RPT)])

    pltpu.sync_copy(src_hbm.at[w, 0], sidx.at[pl.ds(0, 2)])
    pltpu.sync_copy(dst_hbm.at[w, 0], didx.at[pl.ds(0, 2)])

    plsc.subcore_barrier()

    def body(i, carry):
        base = (i % 2) * 2
        nbase = 2 - base
        nxt = lax.min(i + 1, NPAIR - 1)
        cp_si = pltpu.async_copy(src_hbm.at[w, nxt],
                                 sidx.at[pl.ds(nbase, 2)], sem_si)
        cp_di = pltpu.async_copy(dst_hbm.at[w, nxt],
                                 didx.at[pl.ds(nbase, 2)], sem_di)
        cp_a = pltpu.async_copy(x_hbm.at[sidx.at[base]], rows_a, sem_a)
        cp_b = pltpu.async_copy(x_hbm.at[sidx.at[base + 1]], rows_b, sem_b)
        cp_a.wait()
        sc_a = pltpu.async_copy(rows_a, agg.at[didx.at[base]], sem_sa,
                                add=True)
        cp_b.wait()
        sc_b = pltpu.async_copy(rows_b, agg.at[didx.at[base + 1]], sem_sb,
                                add=True)
        sc_a.wait()
        sc_b.wait()
        cp_si.wait()
        cp_di.wait()
        return carry

    lax.fori_loop(0, NPAIR, body, 0)

    plsc.subcore_barrier()

    pltpu.sync_copy(agg.at[pl.ds(s * RPT, RPT)],
                    out_hbm.at[pl.ds(c * NP + s * RPT, RPT)])


def _mlp_block(h, W, b, g, be):
    h = lax.dot_general(h, W, (((1,), (1,)), ((), ())),
                        preferred_element_type=jnp.float32) + b
    m = jnp.mean(h, axis=0, keepdims=True)
    v = jnp.mean((h - m) ** 2, axis=0, keepdims=True)
    h = (h - m) / jnp.sqrt(v + 1e-5) * g + be
    return h * jax.nn.sigmoid(h)


def _tc_mlp_body(p0, p1, x, eps, W1, b1, g1, be1, W2, b2, g2, be2, o):
    h = p0[...] + p1[...] + (1.0 + eps[0, 0]) * x[...]
    h = _mlp_block(h, W1[...], b1[...], g1[...], be1[...])
    h = _mlp_block(h, W2[...], b2[...], g2[...], be2[...])
    o[...] = h


def kernel(x, edge_index, eps, W1, b1, g1, be1, W2, b2, g2, be2):
    src = edge_index[0].astype(jnp.int32)
    dst = edge_index[1].astype(jnp.int32)
    pad = EPAD - E
    src = jnp.concatenate([src, jnp.zeros((pad,), jnp.int32)])
    dst = jnp.concatenate([dst, jnp.full((pad,), N, jnp.int32)])
    src = src.reshape(NW, NPAIR, 2, C)
    dst = dst.reshape(NW, NPAIR, 2, C)
    zeros = jnp.zeros((NP, D), jnp.float32)

    partials = _sc_segment_sum(x, src, dst, zeros)
    p0 = partials[:N]
    p1 = partials[NP:NP + N]

    out = pl.pallas_call(
        _tc_mlp_body,
        out_shape=jax.ShapeDtypeStruct((N, D), jnp.float32),
    )(p0, p1, x, eps.reshape(1, 1),
      W1, b1.reshape(1, D), g1.reshape(1, D), be1.reshape(1, D),
      W2, b2.reshape(1, D), g2.reshape(1, D), be2.reshape(1, D))
    return out


# speedup vs baseline: 3.5756x; 1.1604x over previous
import functools

import jax
import jax.numpy as jnp
from jax import lax
from jax.experimental import pallas as pl
from jax.experimental.pallas import tpu as pltpu
from jax.experimental.pallas import tpu_sc as plsc

N = 10000
E = 320000
D = 128

NT = 16
NW = 32
C = 128
NPAIR0 = 62
NPAIR1 = 18
E0 = NT * NPAIR0 * 2 * C
E1 = NT * NPAIR1 * 2 * C
EPAD = E0 + E1
NP = 10240
RPT = NP // NT

_mesh = plsc.VectorSubcoreMesh(core_axis_name="c", subcore_axis_name="s")


@functools.partial(
    pl.kernel,
    mesh=_mesh,
    out_type=jax.ShapeDtypeStruct((2 * NP, D), jnp.float32),
    scratch_types=[
        pltpu.VMEM((4, C), jnp.int32),
        pltpu.VMEM((4, C), jnp.int32),
        pltpu.VMEM((C, D), jnp.float32),
        pltpu.VMEM((C, D), jnp.float32),
        pltpu.VMEM_SHARED((NP, D), jnp.float32),
        pltpu.SemaphoreType.DMA,
        pltpu.SemaphoreType.DMA,
        pltpu.SemaphoreType.DMA,
        pltpu.SemaphoreType.DMA,
        pltpu.SemaphoreType.DMA,
        pltpu.SemaphoreType.DMA,
    ],
)
def _sc_segment_sum(x_hbm, src0_hbm, dst0_hbm, src1_hbm, dst1_hbm,
                    zero_hbm, out_hbm,
                    sidx, didx, rows_a, rows_b, agg,
                    sem_a, sem_b, sem_si, sem_di, sem_sa, sem_sb):
    c = lax.axis_index("c")
    s = lax.axis_index("s")

    pltpu.sync_copy(zero_hbm.at[pl.ds(s * RPT, RPT)],
                    agg.at[pl.ds(s * RPT, RPT)])

    plsc.subcore_barrier()

    def edge_loop(src_hbm, dst_hbm, npair):
        pltpu.sync_copy(src_hbm.at[s, 0], sidx.at[pl.ds(0, 2)])
        pltpu.sync_copy(dst_hbm.at[s, 0], didx.at[pl.ds(0, 2)])

        def body(i, carry):
            base = (i % 2) * 2
            nbase = 2 - base
            nxt = lax.min(i + 1, npair - 1)
            cp_si = pltpu.async_copy(src_hbm.at[s, nxt],
                                     sidx.at[pl.ds(nbase, 2)], sem_si)
            cp_di = pltpu.async_copy(dst_hbm.at[s, nxt],
                                     didx.at[pl.ds(nbase, 2)], sem_di)
            cp_a = pltpu.async_copy(x_hbm.at[sidx.at[base]], rows_a, sem_a)
            cp_b = pltpu.async_copy(x_hbm.at[sidx.at[base + 1]], rows_b,
                                    sem_b)
            cp_a.wait()
            sc_a = pltpu.async_copy(rows_a, agg.at[didx.at[base]], sem_sa,
                                    add=True)
            cp_b.wait()
            sc_b = pltpu.async_copy(rows_b, agg.at[didx.at[base + 1]],
                                    sem_sb, add=True)
            sc_a.wait()
            sc_b.wait()
            cp_si.wait()
            cp_di.wait()
            return carry

        lax.fori_loop(0, npair, body, 0)

    @pl.when(c == 0)
    def _():
        edge_loop(src0_hbm, dst0_hbm, NPAIR0)

    @pl.when(c == 1)
    def _():
        edge_loop(src1_hbm, dst1_hbm, NPAIR1)

    plsc.subcore_barrier()

    pltpu.sync_copy(agg.at[pl.ds(s * RPT, RPT)],
                    out_hbm.at[pl.ds(c * NP + s * RPT, RPT)])


def _mlp_block(h, W, b, g, be):
    h = lax.dot_general(h, W, (((1,), (1,)), ((), ())),
                        preferred_element_type=jnp.float32) + b
    m = jnp.mean(h, axis=0, keepdims=True)
    v = jnp.mean((h - m) ** 2, axis=0, keepdims=True)
    h = (h - m) / jnp.sqrt(v + 1e-5) * g + be
    return h * jax.nn.sigmoid(h)


def _tc_mlp_body(p0, p1, x, eps, W1, b1, g1, be1, W2, b2, g2, be2, o):
    h = p0[...] + p1[...] + (1.0 + eps[0, 0]) * x[...]
    h = _mlp_block(h, W1[...], b1[...], g1[...], be1[...])
    h = _mlp_block(h, W2[...], b2[...], g2[...], be2[...])
    o[...] = h


def kernel(x, edge_index, eps, W1, b1, g1, be1, W2, b2, g2, be2):
    src = edge_index[0].astype(jnp.int32)
    dst = edge_index[1].astype(jnp.int32)
    pad = EPAD - E
    src = jnp.concatenate([src, jnp.zeros((pad,), jnp.int32)])
    dst = jnp.concatenate([dst, jnp.full((pad,), N, jnp.int32)])
    src0 = src[:E0].reshape(NT, NPAIR0, 2, C)
    dst0 = dst[:E0].reshape(NT, NPAIR0, 2, C)
    src1 = src[E0:].reshape(NT, NPAIR1, 2, C)
    dst1 = dst[E0:].reshape(NT, NPAIR1, 2, C)
    zeros = jnp.zeros((NP, D), jnp.float32)

    partials = _sc_segment_sum(x, src0, dst0, src1, dst1, zeros)
    p0 = partials[:N]
    p1 = partials[NP:NP + N]

    out = pl.pallas_call(
        _tc_mlp_body,
        out_shape=jax.ShapeDtypeStruct((N, D), jnp.float32),
    )(p0, p1, x, eps.reshape(1, 1),
      W1, b1.reshape(1, D), g1.reshape(1, D), be1.reshape(1, D),
      W2, b2.reshape(1, D), g2.reshape(1, D), be2.reshape(1, D))
    return out


# speedup vs baseline: 8.6785x; 2.4272x over previous
import functools

import jax
import jax.numpy as jnp
from jax import lax
from jax.experimental import pallas as pl
from jax.experimental.pallas import tpu as pltpu
from jax.experimental.pallas import tpu_sc as plsc

N = 10000
E = 320000
D = 128

NT = 16
NW = 32
C = 128
NPAIR = 40
EPAD = NW * NPAIR * 2 * C
NP = 10240
RPT = NP // NT

_mesh = plsc.VectorSubcoreMesh(core_axis_name="c", subcore_axis_name="s")


@functools.partial(
    pl.kernel,
    mesh=_mesh,
    out_type=jax.ShapeDtypeStruct((2 * NP, D), jnp.float32),
    scratch_types=[
        pltpu.VMEM((4, C), jnp.int32),
        pltpu.VMEM((4, C), jnp.int32),
        pltpu.VMEM((C, D), jnp.float32),
        pltpu.VMEM((C, D), jnp.float32),
        pltpu.VMEM_SHARED((NP, D), jnp.float32),
        pltpu.SemaphoreType.DMA,
        pltpu.SemaphoreType.DMA,
        pltpu.SemaphoreType.DMA,
        pltpu.SemaphoreType.DMA,
        pltpu.SemaphoreType.DMA,
        pltpu.SemaphoreType.DMA,
    ],
)
def _sc_segment_sum(x_hbm, src_hbm, dst_hbm, zero_hbm, out_hbm,
                    sidx, didx, rows_a, rows_b, agg,
                    sem_a, sem_b, sem_si, sem_di, sem_sa, sem_sb):
    c = lax.axis_index("c")
    s = lax.axis_index("s")
    w = c * NT + s

    pltpu.sync_copy(zero_hbm.at[pl.ds(s * RPT, RPT)],
                    agg.at[pl.ds(s * RPT, RPT)])

    plsc.subcore_barrier()

    pltpu.sync_copy(src_hbm.at[w, 0], sidx.at[pl.ds(0, 2)])
    pltpu.sync_copy(dst_hbm.at[w, 0], didx.at[pl.ds(0, 2)])

    def body(i, carry):
        base = (i % 2) * 2
        nbase = 2 - base
        nxt = lax.min(i + 1, NPAIR - 1)
        cp_si = pltpu.async_copy(src_hbm.at[w, nxt],
                                 sidx.at[pl.ds(nbase, 2)], sem_si)
        cp_di = pltpu.async_copy(dst_hbm.at[w, nxt],
                                 didx.at[pl.ds(nbase, 2)], sem_di)
        cp_a = pltpu.async_copy(x_hbm.at[sidx.at[base]], rows_a, sem_a)
        cp_b = pltpu.async_copy(x_hbm.at[sidx.at[base + 1]], rows_b, sem_b)
        cp_a.wait()
        sc_a = pltpu.async_copy(rows_a, agg.at[didx.at[base]], sem_sa,
                                add=True)
        cp_b.wait()
        sc_b = pltpu.async_copy(rows_b, agg.at[didx.at[base + 1]],
                                sem_sb, add=True)
        sc_a.wait()
        sc_b.wait()
        cp_si.wait()
        cp_di.wait()
        return carry

    lax.fori_loop(0, NPAIR, body, 0)

    plsc.subcore_barrier()

    pltpu.sync_copy(agg.at[pl.ds(s * RPT, RPT)],
                    out_hbm.at[pl.ds(c * NP + s * RPT, RPT)])


def _mlp_block(h, W, b, g, be):
    h = lax.dot_general(h, W, (((1,), (1,)), ((), ())),
                        preferred_element_type=jnp.float32) + b
    m = jnp.mean(h, axis=0, keepdims=True)
    v = jnp.mean((h - m) ** 2, axis=0, keepdims=True)
    h = (h - m) / jnp.sqrt(v + 1e-5) * g + be
    return h * jax.nn.sigmoid(h)


def _tc_mlp_body(p0, p1, x, eps, W1, b1, g1, be1, W2, b2, g2, be2, o):
    h = p0[...] + p1[...] + (1.0 + eps[0, 0]) * x[...]
    h = _mlp_block(h, W1[...], b1[...], g1[...], be1[...])
    h = _mlp_block(h, W2[...], b2[...], g2[...], be2[...])
    o[...] = h


def kernel(x, edge_index, eps, W1, b1, g1, be1, W2, b2, g2, be2):
    src = edge_index[0].astype(jnp.int32)
    dst = edge_index[1].astype(jnp.int32)
    pad = EPAD - E
    iot = jnp.arange(pad, dtype=jnp.int32)
    src = jnp.concatenate([src, iot % N])
    dst = jnp.concatenate([dst, N + iot % (NP - N)])
    src = src.reshape(NW, NPAIR, 2, C)
    dst = dst.reshape(NW, NPAIR, 2, C)
    zeros = jnp.zeros((NP, D), jnp.float32)

    partials = _sc_segment_sum(x, src, dst, zeros)
    p0 = partials[:N]
    p1 = partials[NP:NP + N]

    out = pl.pallas_call(
        _tc_mlp_body,
        out_shape=jax.ShapeDtypeStruct((N, D), jnp.float32),
    )(p0, p1, x, eps.reshape(1, 1),
      W1, b1.reshape(1, D), g1.reshape(1, D), be1.reshape(1, D),
      W2, b2.reshape(1, D), g2.reshape(1, D), be2.reshape(1, D))
    return out


# speedup vs baseline: 9.0242x; 1.0398x over previous
import functools

import jax
import jax.numpy as jnp
from jax import lax
from jax.experimental import pallas as pl
from jax.experimental.pallas import tpu as pltpu
from jax.experimental.pallas import tpu_sc as plsc

N = 10000
E = 320000
D = 128

NT = 16
NW = 32
C = 128
NPAIR = 40
EPAD = NW * NPAIR * 2 * C
NP = 10240
RPT = NP // NT

_mesh = plsc.VectorSubcoreMesh(core_axis_name="c", subcore_axis_name="s")


@functools.partial(
    pl.kernel,
    mesh=_mesh,
    out_type=jax.ShapeDtypeStruct((2 * NP, D), jnp.float32),
    scratch_types=[
        pltpu.VMEM((4, C), jnp.int32),
        pltpu.VMEM((4, C), jnp.int32),
        pltpu.VMEM((C, D), jnp.float32),
        pltpu.VMEM((C, D), jnp.float32),
        pltpu.VMEM_SHARED((NP, D), jnp.float32),
        pltpu.SemaphoreType.DMA,
        pltpu.SemaphoreType.DMA,
        pltpu.SemaphoreType.DMA,
        pltpu.SemaphoreType.DMA,
        pltpu.SemaphoreType.DMA,
        pltpu.SemaphoreType.DMA,
    ],
)
def _sc_segment_sum(x_hbm, src_hbm, dst_hbm, zero_hbm, out_hbm,
                    sidx, didx, rows_a, rows_b, agg,
                    sem_a, sem_b, sem_si, sem_di, sem_sa, sem_sb):
    c = lax.axis_index("c")
    s = lax.axis_index("s")
    w = c * NT + s

    cp_z = pltpu.async_copy(zero_hbm.at[pl.ds(s * RPT, RPT)],
                            agg.at[pl.ds(s * RPT, RPT)], sem_sa)

    pltpu.sync_copy(src_hbm.at[w, 0], sidx.at[pl.ds(0, 2)])
    pltpu.sync_copy(dst_hbm.at[w, 0], didx.at[pl.ds(0, 2)])

    cp_z.wait()
    plsc.subcore_barrier()

    def body(i, carry):
        base = (i % 2) * 2
        nbase = 2 - base
        nxt = lax.min(i + 1, NPAIR - 1)
        cp_si = pltpu.async_copy(src_hbm.at[w, nxt],
                                 sidx.at[pl.ds(nbase, 2)], sem_si)
        cp_di = pltpu.async_copy(dst_hbm.at[w, nxt],
                                 didx.at[pl.ds(nbase, 2)], sem_di)
        cp_a = pltpu.async_copy(x_hbm.at[sidx.at[base]], rows_a, sem_a)
        cp_b = pltpu.async_copy(x_hbm.at[sidx.at[base + 1]], rows_b, sem_b)
        cp_a.wait()
        sc_a = pltpu.async_copy(rows_a, agg.at[didx.at[base]], sem_sa,
                                add=True)
        cp_b.wait()
        sc_b = pltpu.async_copy(rows_b, agg.at[didx.at[base + 1]],
                                sem_sb, add=True)
        sc_a.wait()
        sc_b.wait()
        cp_si.wait()
        cp_di.wait()
        return carry

    lax.fori_loop(0, NPAIR, body, 0)

    plsc.subcore_barrier()

    pltpu.sync_copy(agg.at[pl.ds(s * RPT, RPT)],
                    out_hbm.at[pl.ds(c * NP + s * RPT, RPT)])


def _mlp_block(h, W, b, g, be):
    h = lax.dot_general(h, W, (((1,), (1,)), ((), ())),
                        preferred_element_type=jnp.float32) + b
    m = jnp.mean(h, axis=0, keepdims=True)
    v = jnp.mean((h - m) ** 2, axis=0, keepdims=True)
    h = (h - m) / jnp.sqrt(v + 1e-5) * g + be
    return h * jax.nn.sigmoid(h)


def _tc_mlp_body(parts, x, eps, W1, b1, g1, be1, W2, b2, g2, be2, o):
    h = (parts[pl.ds(0, N), :] + parts[pl.ds(NP, N), :]
         + (1.0 + eps[0, 0]) * x[...])
    h = _mlp_block(h, W1[...], b1[...], g1[...], be1[...])
    h = _mlp_block(h, W2[...], b2[...], g2[...], be2[...])
    o[...] = h


def kernel(x, edge_index, eps, W1, b1, g1, be1, W2, b2, g2, be2):
    src = edge_index[0].astype(jnp.int32)
    dst = edge_index[1].astype(jnp.int32)
    pad = EPAD - E
    iot = jnp.arange(pad, dtype=jnp.int32)
    src = jnp.concatenate([src, iot % N])
    dst = jnp.concatenate([dst, N + iot % (NP - N)])
    src = src.reshape(NW, NPAIR, 2, C)
    dst = dst.reshape(NW, NPAIR, 2, C)
    zeros = jnp.zeros((NP, D), jnp.float32)

    partials = _sc_segment_sum(x, src, dst, zeros)

    out = pl.pallas_call(
        _tc_mlp_body,
        out_shape=jax.ShapeDtypeStruct((N, D), jnp.float32),
    )(partials, x, eps.reshape(1, 1),
      W1, b1.reshape(1, D), g1.reshape(1, D), be1.reshape(1, D),
      W2, b2.reshape(1, D), g2.reshape(1, D), be2.reshape(1, D))
    return out


# speedup vs baseline: 10.4365x; 1.1565x over previous
import functools

import jax
import jax.numpy as jnp
from jax import lax
from jax.experimental import pallas as pl
from jax.experimental.pallas import tpu as pltpu
from jax.experimental.pallas import tpu_sc as plsc

N = 10000
E = 320000
D = 128

NT = 16
NW = 32
C = 64
NCH = 160
NPAIR = NCH // 2
EPAD = NW * NCH * C
NP = 10240
RPT = NP // NT

_mesh = plsc.VectorSubcoreMesh(core_axis_name="c", subcore_axis_name="s")


@functools.partial(
    pl.kernel,
    mesh=_mesh,
    out_type=jax.ShapeDtypeStruct((2 * NP, D), jnp.float32),
    scratch_types=[
        pltpu.VMEM((4, 2 * C), jnp.int32),
        pltpu.VMEM((4, 2 * C), jnp.int32),
        pltpu.VMEM((C, D), jnp.float32),
        pltpu.VMEM((C, D), jnp.float32),
        pltpu.VMEM((C, D), jnp.float32),
        pltpu.VMEM((C, D), jnp.float32),
        pltpu.VMEM_SHARED((NP, D), jnp.float32),
        pltpu.SemaphoreType.DMA,
        pltpu.SemaphoreType.DMA,
        pltpu.SemaphoreType.DMA,
        pltpu.SemaphoreType.DMA,
        pltpu.SemaphoreType.DMA,
        pltpu.SemaphoreType.DMA,
    ],
)
def _sc_segment_sum(x_hbm, src_hbm, dst_hbm, zero_hbm, out_hbm,
                    sidx, didx, rows0, rows1, rows2, rows3, agg,
                    sem_g0, sem_g1, sem_s0, sem_s1, sem_si, sem_di):
    c = lax.axis_index("c")
    s = lax.axis_index("s")
    w = c * NT + s

    def gather(p, h, buf, sem):
        return pltpu.async_copy(
            x_hbm.at[sidx.at[p % 4, pl.ds(h * C, C)]], buf, sem)

    def scatter(buf, p, h, sem):
        return pltpu.async_copy(
            buf, agg.at[didx.at[p % 4, pl.ds(h * C, C)]], sem, add=True)

    def pf(p, slot):
        return (pltpu.async_copy(src_hbm.at[w, p], sidx.at[slot], sem_si),
                pltpu.async_copy(dst_hbm.at[w, p], didx.at[slot], sem_di))

    def drain_g(sem, buf):
        pltpu.make_async_copy(zero_hbm.at[pl.ds(0, C)], buf, sem).wait()

    def drain_s(sem, buf):
        pltpu.make_async_copy(zero_hbm.at[pl.ds(0, C)], buf, sem).wait()

    def drain_pf():
        pltpu.make_async_copy(src_hbm.at[w, 0], sidx.at[0], sem_si).wait()
        pltpu.make_async_copy(dst_hbm.at[w, 0], didx.at[0], sem_di).wait()

    cp_z = pltpu.async_copy(zero_hbm.at[pl.ds(s * RPT, RPT)],
                            agg.at[pl.ds(s * RPT, RPT)], sem_s0)

    pltpu.sync_copy(src_hbm.at[w, 0], sidx.at[0])
    pltpu.sync_copy(dst_hbm.at[w, 0], didx.at[0])
    pf_s = pltpu.async_copy(src_hbm.at[w, 1], sidx.at[1], sem_si)
    pf_d = pltpu.async_copy(dst_hbm.at[w, 1], didx.at[1], sem_di)

    cp_z.wait()
    plsc.subcore_barrier()

    g_a = gather(0, 0, rows0, sem_g0)
    g_b = gather(0, 1, rows1, sem_g1)

    g_a.wait()
    scatter(rows0, 0, 0, sem_s0)
    pf_s.wait()
    pf_d.wait()
    gather(1, 0, rows2, sem_g0)
    g_b.wait()
    scatter(rows1, 0, 1, sem_s1)
    gather(1, 1, rows3, sem_g1)
    pf(2, 2)

    def body(q, carry):
        p = 2 * q + 1
        drain_g(sem_g0, rows2)
        drain_s(sem_s0, rows0)
        s_a = scatter(rows2, p, 0, sem_s0)
        drain_pf()
        g_a = gather(p + 1, 0, rows0, sem_g0)
        drain_g(sem_g1, rows3)
        drain_s(sem_s1, rows1)
        s_b = scatter(rows3, p, 1, sem_s1)
        g_b = gather(p + 1, 1, rows1, sem_g1)
        pf_a, pf_b = pf(lax.min(p + 2, NPAIR - 1), (p + 2) % 4)
        g_a.wait()
        s_a.wait()
        s_c = scatter(rows0, p + 1, 0, sem_s0)
        pf_a.wait()
        pf_b.wait()
        gather(lax.min(p + 2, NPAIR - 1), 0, rows2, sem_g0)
        g_b.wait()
        s_b.wait()
        scatter(rows1, p + 1, 1, sem_s1)
        gather(lax.min(p + 2, NPAIR - 1), 1, rows3, sem_g1)
        pf(lax.min(p + 3, NPAIR - 1), (p + 3) % 4)
        return carry

    lax.fori_loop(0, (NPAIR - 2) // 2, body, 0)

    drain_g(sem_g0, rows2)
    drain_s(sem_s0, rows0)
    s_a = scatter(rows2, NPAIR - 1, 0, sem_s0)
    drain_g(sem_g1, rows3)
    drain_s(sem_s1, rows1)
    s_b = scatter(rows3, NPAIR - 1, 1, sem_s1)
    drain_pf()
    s_a.wait()
    s_b.wait()

    plsc.subcore_barrier()

    pltpu.sync_copy(agg.at[pl.ds(s * RPT, RPT)],
                    out_hbm.at[pl.ds(c * NP + s * RPT, RPT)])


def _mlp_block(h, W, b, g, be):
    h = lax.dot_general(h, W, (((1,), (1,)), ((), ())),
                        preferred_element_type=jnp.float32) + b
    m = jnp.mean(h, axis=0, keepdims=True)
    v = jnp.mean((h - m) ** 2, axis=0, keepdims=True)
    h = (h - m) / jnp.sqrt(v + 1e-5) * g + be
    return h * jax.nn.sigmoid(h)


def _tc_mlp_body(parts, x, eps, W1, b1, g1, be1, W2, b2, g2, be2, o):
    h = (parts[pl.ds(0, N), :] + parts[pl.ds(NP, N), :]
         + (1.0 + eps[0, 0]) * x[...])
    h = _mlp_block(h, W1[...], b1[...], g1[...], be1[...])
    h = _mlp_block(h, W2[...], b2[...], g2[...], be2[...])
    o[...] = h


def kernel(x, edge_index, eps, W1, b1, g1, be1, W2, b2, g2, be2):
    src = edge_index[0].astype(jnp.int32)
    dst = edge_index[1].astype(jnp.int32)
    pad = EPAD - E
    iot = jnp.arange(pad, dtype=jnp.int32)
    src = jnp.concatenate([src, iot % N])
    dst = jnp.concatenate([dst, N + iot % (NP - N)])
    src = src.reshape(NW, NPAIR, 2 * C)
    dst = dst.reshape(NW, NPAIR, 2 * C)
    zeros = jnp.zeros((NP, D), jnp.float32)

    partials = _sc_segment_sum(x, src, dst, zeros)

    out = pl.pallas_call(
        _tc_mlp_body,
        out_shape=jax.ShapeDtypeStruct((N, D), jnp.float32),
    )(partials, x, eps.reshape(1, 1),
      W1, b1.reshape(1, D), g1.reshape(1, D), be1.reshape(1, D),
      W2, b2.reshape(1, D), g2.reshape(1, D), be2.reshape(1, D))
    return out


# speedup vs baseline: 10.4911x; 1.0052x over previous
import functools

import jax
import jax.numpy as jnp
from jax import lax
from jax.experimental import pallas as pl
from jax.experimental.pallas import tpu as pltpu
from jax.experimental.pallas import tpu_sc as plsc

N = 10000
E = 320000
D = 128

NT = 16
NW = 32
C = 64
NCH = 160
NPAIR = NCH // 2
EPAD = NW * NCH * C
NP = 10240
RPT = NP // NT

_mesh = plsc.VectorSubcoreMesh(core_axis_name="c", subcore_axis_name="s")


@functools.partial(
    pl.kernel,
    mesh=_mesh,
    out_type=jax.ShapeDtypeStruct((2 * N, D), jnp.float32),
    scratch_types=[
        pltpu.VMEM((4, 2 * C), jnp.int32),
        pltpu.VMEM((4, 2 * C), jnp.int32),
        pltpu.VMEM((C, D), jnp.float32),
        pltpu.VMEM((C, D), jnp.float32),
        pltpu.VMEM((C, D), jnp.float32),
        pltpu.VMEM((C, D), jnp.float32),
        pltpu.VMEM_SHARED((NP, D), jnp.float32),
        pltpu.SemaphoreType.DMA,
        pltpu.SemaphoreType.DMA,
        pltpu.SemaphoreType.DMA,
        pltpu.SemaphoreType.DMA,
        pltpu.SemaphoreType.DMA,
        pltpu.SemaphoreType.DMA,
    ],
)
def _sc_segment_sum(x_hbm, src_hbm, dst_hbm, zero_hbm, out_hbm,
                    sidx, didx, rows0, rows1, rows2, rows3, agg,
                    sem_g0, sem_g1, sem_s0, sem_s1, sem_si, sem_di):
    c = lax.axis_index("c")
    s = lax.axis_index("s")
    w = c * NT + s

    def gather(p, h, buf, sem):
        return pltpu.async_copy(
            x_hbm.at[sidx.at[p % 4, pl.ds(h * C, C)]], buf, sem)

    def scatter(buf, p, h, sem):
        return pltpu.async_copy(
            buf, agg.at[didx.at[p % 4, pl.ds(h * C, C)]], sem, add=True)

    def pf(p, slot):
        return (pltpu.async_copy(src_hbm.at[w, p], sidx.at[slot], sem_si),
                pltpu.async_copy(dst_hbm.at[w, p], didx.at[slot], sem_di))

    def drain_g(sem, buf):
        pltpu.make_async_copy(zero_hbm.at[pl.ds(0, C)], buf, sem).wait()

    def drain_s(sem, buf):
        pltpu.make_async_copy(zero_hbm.at[pl.ds(0, C)], buf, sem).wait()

    def drain_pf():
        pltpu.make_async_copy(src_hbm.at[w, 0], sidx.at[0], sem_si).wait()
        pltpu.make_async_copy(dst_hbm.at[w, 0], didx.at[0], sem_di).wait()

    cp_z = pltpu.async_copy(zero_hbm,
                            agg.at[pl.ds(s * RPT, RPT)], sem_s0)

    pltpu.sync_copy(src_hbm.at[w, 0], sidx.at[0])
    pltpu.sync_copy(dst_hbm.at[w, 0], didx.at[0])
    pf_s = pltpu.async_copy(src_hbm.at[w, 1], sidx.at[1], sem_si)
    pf_d = pltpu.async_copy(dst_hbm.at[w, 1], didx.at[1], sem_di)

    cp_z.wait()
    plsc.subcore_barrier()

    g_a = gather(0, 0, rows0, sem_g0)
    g_b = gather(0, 1, rows1, sem_g1)

    g_a.wait()
    scatter(rows0, 0, 0, sem_s0)
    pf_s.wait()
    pf_d.wait()
    gather(1, 0, rows2, sem_g0)
    g_b.wait()
    scatter(rows1, 0, 1, sem_s1)
    gather(1, 1, rows3, sem_g1)
    pf(2, 2)

    def body(q, carry):
        p = 2 * q + 1
        drain_g(sem_g0, rows2)
        drain_s(sem_s0, rows0)
        s_a = scatter(rows2, p, 0, sem_s0)
        drain_pf()
        g_a = gather(p + 1, 0, rows0, sem_g0)
        drain_g(sem_g1, rows3)
        drain_s(sem_s1, rows1)
        s_b = scatter(rows3, p, 1, sem_s1)
        g_b = gather(p + 1, 1, rows1, sem_g1)
        pf_a, pf_b = pf(lax.min(p + 2, NPAIR - 1), (p + 2) % 4)
        g_a.wait()
        s_a.wait()
        s_c = scatter(rows0, p + 1, 0, sem_s0)
        pf_a.wait()
        pf_b.wait()
        gather(lax.min(p + 2, NPAIR - 1), 0, rows2, sem_g0)
        g_b.wait()
        s_b.wait()
        scatter(rows1, p + 1, 1, sem_s1)
        gather(lax.min(p + 2, NPAIR - 1), 1, rows3, sem_g1)
        pf(lax.min(p + 3, NPAIR - 1), (p + 3) % 4)
        return carry

    lax.fori_loop(0, (NPAIR - 2) // 2, body, 0)

    drain_g(sem_g0, rows2)
    drain_s(sem_s0, rows0)
    s_a = scatter(rows2, NPAIR - 1, 0, sem_s0)
    drain_g(sem_g1, rows3)
    drain_s(sem_s1, rows1)
    s_b = scatter(rows3, NPAIR - 1, 1, sem_s1)
    drain_pf()
    s_a.wait()
    s_b.wait()

    plsc.subcore_barrier()

    off = lax.min(s * RPT, N - RPT)
    pltpu.sync_copy(agg.at[pl.ds(off, RPT)],
                    out_hbm.at[pl.ds(c * N + off, RPT)])


def _mlp_block(h, W, b, g, be):
    h = lax.dot_general(h, W, (((1,), (1,)), ((), ())),
                        preferred_element_type=jnp.float32) + b
    m = jnp.mean(h, axis=0, keepdims=True)
    v = jnp.mean((h - m) ** 2, axis=0, keepdims=True)
    h = (h - m) / jnp.sqrt(v + 1e-5) * g + be
    return h * jax.nn.sigmoid(h)


def _tc_mlp_body(parts, x, eps, W1, b1, g1, be1, W2, b2, g2, be2, o):
    h = (parts[pl.ds(0, N), :] + parts[pl.ds(N, N), :]
         + (1.0 + eps[0, 0]) * x[...])
    h = _mlp_block(h, W1[...], b1[...], g1[...], be1[...])
    h = _mlp_block(h, W2[...], b2[...], g2[...], be2[...])
    o[...] = h


def kernel(x, edge_index, eps, W1, b1, g1, be1, W2, b2, g2, be2):
    src = edge_index[0].astype(jnp.int32)
    dst = edge_index[1].astype(jnp.int32)
    pad = EPAD - E
    iot = jnp.arange(pad, dtype=jnp.int32)
    src = jnp.concatenate([src, iot % N])
    dst = jnp.concatenate([dst, N + iot % (NP - N)])
    src = src.reshape(NW, NPAIR, 2 * C)
    dst = dst.reshape(NW, NPAIR, 2 * C)
    zeros = jnp.zeros((RPT, D), jnp.float32)

    partials = _sc_segment_sum(x, src, dst, zeros)

    out = pl.pallas_call(
        _tc_mlp_body,
        out_shape=jax.ShapeDtypeStruct((N, D), jnp.float32),
    )(partials, x, eps.reshape(1, 1),
      W1, b1.reshape(1, D), g1.reshape(1, D), be1.reshape(1, D),
      W2, b2.reshape(1, D), g2.reshape(1, D), be2.reshape(1, D))
    return out
